# Initial kernel scaffold; baseline (speedup 1.0000x reference)
import jax
import jax.numpy as jnp
from jax.experimental import pallas as pl


def kernel(ins_masks_masks, ins_masks_scores, ins_masks_class_ids, sem_masks):
    raise NotImplementedError("write your pallas kernel here")


# speedup vs baseline: 6.8380x; 6.8380x over previous
import jax
import jax.numpy as jnp
from jax.experimental import pallas as pl
from jax.experimental.pallas import tpu as pltpu

_INSTANCE_OFFSET = 1000
_NUM_THINGS = 80
_NUM_STUFF = 53
_IGNORE = 53
_STUFF_AREA_THR = 4096
_THING_CONF_THR = 0.5


def _fusion_body(sind_ref, score_ref, cls_ref, mask_ref, sem_ref, out_ref,
                 insid_ref):
    b = pl.program_id(0)
    i = pl.program_id(1)
    n = pl.num_programs(1)

    @pl.when(i == 0)
    def _init():
        out_ref[...] = jnp.zeros(out_ref.shape, out_ref.dtype)
        insid_ref[0] = jnp.int32(1)

    mask = mask_ref[0, 0]
    pan = out_ref[0]
    mask_i32 = mask.astype(jnp.int32)
    mask_area = jnp.sum(mask_i32)
    occupied = pan != 0
    inter_area = jnp.sum(jnp.where(occupied, mask_i32, 0))
    score = score_ref[b, i]
    keep = jnp.logical_and(
        jnp.logical_and(score >= _THING_CONF_THR, mask_area > 0),
        2 * inter_area <= mask_area,
    )
    ins_id = insid_ref[0]
    label = cls_ref[b, i] + ins_id * _INSTANCE_OFFSET
    paint = jnp.logical_and(jnp.logical_and(mask, jnp.logical_not(occupied)), keep)
    out_ref[0] = jnp.where(paint, label, pan)
    insid_ref[0] = ins_id + keep.astype(jnp.int32)

    @pl.when(i == n - 1)
    def _stuff():
        pan2 = out_ref[0]
        sem = jnp.where(pan2 > 0, jnp.int32(_IGNORE), sem_ref[0])
        res = pan2
        for c in range(_NUM_STUFF):
            m = sem == c
            cnt = jnp.sum(m.astype(jnp.int32))
            res = jnp.where(
                jnp.logical_and(m, cnt >= _STUFF_AREA_THR),
                jnp.int32(c + _NUM_THINGS),
                res,
            )
        out_ref[0] = res


def kernel(ins_masks_masks, ins_masks_scores, ins_masks_class_ids, sem_masks):
    B, N, H, W = ins_masks_masks.shape
    sorted_inds = jnp.argsort(-ins_masks_scores, axis=1).astype(jnp.int32)
    s_scores = jnp.take_along_axis(ins_masks_scores, sorted_inds, axis=1)
    s_cls = jnp.take_along_axis(
        ins_masks_class_ids.astype(jnp.int32), sorted_inds, axis=1)

    grid_spec = pltpu.PrefetchScalarGridSpec(
        num_scalar_prefetch=3,
        grid=(B, N),
        in_specs=[
            pl.BlockSpec((1, 1, H, W),
                         lambda b, i, sind, sc, cl: (b, sind[b, i], 0, 0)),
            pl.BlockSpec((1, H, W), lambda b, i, sind, sc, cl: (b, 0, 0)),
        ],
        out_specs=pl.BlockSpec((1, H, W), lambda b, i, sind, sc, cl: (b, 0, 0)),
        scratch_shapes=[pltpu.SMEM((1,), jnp.int32)],
    )
    return pl.pallas_call(
        _fusion_body,
        grid_spec=grid_spec,
        out_shape=jax.ShapeDtypeStruct((B, H, W), jnp.int32),
    )(sorted_inds, s_scores, s_cls, ins_masks_masks,
      sem_masks.astype(jnp.int32))


# speedup vs baseline: 11.1052x; 1.6240x over previous
import jax
import jax.numpy as jnp
from jax.experimental import pallas as pl
from jax.experimental.pallas import tpu as pltpu

_INSTANCE_OFFSET = 1000
_NUM_THINGS = 80
_NUM_STUFF = 53
_IGNORE = 53
_STUFF_AREA_THR = 4096
_THING_CONF_THR = 0.5


def _fusion_body(gind_ref, score_ref, cls_ref, mask_ref, sem_ref, out_ref,
                 insid_ref, occ_ref):
    del gind_ref
    b = pl.program_id(0)
    i = pl.program_id(1)
    n = pl.num_programs(1)

    @pl.when(i == 0)
    def _init():
        occ_ref[...] = jnp.zeros(occ_ref.shape, occ_ref.dtype)
        insid_ref[0] = jnp.int32(1)

    @pl.when(score_ref[b, i] >= _THING_CONF_THR)
    def _instance():
        mask = mask_ref[0, 0]
        occ = occ_ref[...]
        free = jnp.logical_and(mask, jnp.logical_not(occ))
        mask_area = jnp.sum(mask.astype(jnp.int32))
        free_area = jnp.sum(free.astype(jnp.int32))
        inter_area = mask_area - free_area
        keep = jnp.logical_and(mask_area > 0, 2 * inter_area <= mask_area)

        @pl.when(keep)
        def _paint():
            ins_id = insid_ref[0]
            label = cls_ref[b, i] + ins_id * _INSTANCE_OFFSET

            @pl.when(ins_id == 1)
            def _first():
                out_ref[0] = jnp.where(free, label, 0)

            @pl.when(ins_id != 1)
            def _rest():
                out_ref[0] = jnp.where(free, label, out_ref[0])

            occ_ref[...] = jnp.logical_or(occ, mask)
            insid_ref[0] = ins_id + 1

    @pl.when(i == n - 1)
    def _stuff():
        covered = occ_ref[...]

        @pl.when(insid_ref[0] == 1)
        def _blank():
            out_ref[0] = jnp.zeros(out_ref.shape[1:], out_ref.dtype)

        pan = out_ref[0]
        sem = jnp.where(covered, jnp.int32(_IGNORE), sem_ref[0])
        lo = jnp.int32(0)
        hi = jnp.int32(0)
        for c in range(_NUM_STUFF):
            ok = (jnp.sum((sem == c).astype(jnp.int32))
                  >= _STUFF_AREA_THR).astype(jnp.int32)
            if c < 32:
                lo = lo + (ok << c)
            else:
                hi = hi + (ok << (c - 32))
        word = jnp.where(sem < 32, lo, hi)
        shift = jnp.where(sem < 32, sem, sem - 32)
        okpix = ((word >> shift) & 1) == 1
        out_ref[0] = jnp.where(covered, pan,
                               jnp.where(okpix, sem + _NUM_THINGS, 0))


def kernel(ins_masks_masks, ins_masks_scores, ins_masks_class_ids, sem_masks):
    B, N, H, W = ins_masks_masks.shape
    sorted_inds = jnp.argsort(-ins_masks_scores, axis=1).astype(jnp.int32)
    s_scores = jnp.take_along_axis(ins_masks_scores, sorted_inds, axis=1)
    s_cls = jnp.take_along_axis(
        ins_masks_class_ids.astype(jnp.int32), sorted_inds, axis=1)
    k = jnp.sum((s_scores >= _THING_CONF_THR).astype(jnp.int32), axis=1)
    eff = jnp.minimum(jnp.arange(N, dtype=jnp.int32)[None, :],
                      jnp.maximum(k[:, None] - 1, 0))
    g_inds = jnp.take_along_axis(sorted_inds, eff, axis=1)

    grid_spec = pltpu.PrefetchScalarGridSpec(
        num_scalar_prefetch=3,
        grid=(B, N),
        in_specs=[
            pl.BlockSpec((1, 1, H, W),
                         lambda b, i, gind, sc, cl: (b, gind[b, i], 0, 0)),
            pl.BlockSpec((1, H, W), lambda b, i, gind, sc, cl: (b, 0, 0)),
        ],
        out_specs=pl.BlockSpec((1, H, W), lambda b, i, gind, sc, cl: (b, 0, 0)),
        scratch_shapes=[
            pltpu.SMEM((1,), jnp.int32),
            pltpu.VMEM((H, W), jnp.bool_),
        ],
    )
    return pl.pallas_call(
        _fusion_body,
        grid_spec=grid_spec,
        out_shape=jax.ShapeDtypeStruct((B, H, W), jnp.int32),
    )(g_inds, s_scores, s_cls, ins_masks_masks,
      sem_masks.astype(jnp.int32))
